# baseline (speedup 1.0000x reference)
import jax
import jax.numpy as jnp
from jax.experimental import pallas as pl
from jax.experimental.pallas import tpu as pltpu

N = 4096
D = 256
R = 3
TILE = 128
JSPLIT = 2
JW = N // JSPLIT


def _fused_body(feature_ref, adj_ref, w_ref, out_ref, h_ref):
    i = pl.program_id(0)
    j = pl.program_id(1)

    @pl.when((i == 0) & (j == 0))
    def _compute_h():
        h_ref[...] = jnp.tanh(
            jnp.dot(feature_ref[...], w_ref[...],
                    preferred_element_type=jnp.float32))

    a = adj_ref[0] + adj_ref[1] + adj_ref[2]
    part = jnp.dot(a, h_ref[pl.ds(j * JW, JW), :],
                   preferred_element_type=jnp.float32)

    @pl.when(j == 0)
    def _init():
        out_ref[...] = part

    @pl.when(j != 0)
    def _acc():
        out_ref[...] += part


@jax.jit
def kernel(feature, same_type_adj, W, b):
    del b
    grid = (N // TILE, JSPLIT)
    return pl.pallas_call(
        _fused_body,
        grid=grid,
        in_specs=[
            pl.BlockSpec((N, D), lambda i, j: (0, 0)),
            pl.BlockSpec((R, TILE, JW), lambda i, j: (0, i, j)),
            pl.BlockSpec((D, D), lambda i, j: (0, 0)),
        ],
        out_specs=pl.BlockSpec((TILE, D), lambda i, j: (i, 0)),
        out_shape=jax.ShapeDtypeStruct((N, D), jnp.float32),
        scratch_shapes=[pltpu.VMEM((N, D), jnp.float32)],
    )(feature, same_type_adj, W)


# speedup vs baseline: 1.2872x; 1.2872x over previous
import jax
import jax.numpy as jnp
from jax.experimental import pallas as pl
from jax.experimental.pallas import tpu as pltpu

N = 4096
D = 256
R = 3
ROW_TILE = 128


def _fused_body(feature_ref, adj_ref, w_ref, out_ref, h_ref):
    i = pl.program_id(0)

    @pl.when(i == 0)
    def _compute_h():
        h_ref[...] = jnp.tanh(
            jnp.dot(feature_ref[...], w_ref[...],
                    preferred_element_type=jnp.float32))

    a = adj_ref[0] + adj_ref[1] + adj_ref[2]
    out_ref[...] = jnp.dot(a, h_ref[...], preferred_element_type=jnp.float32)


@jax.jit
def kernel(feature, same_type_adj, W, b):
    del b
    grid = (N // ROW_TILE,)
    return pl.pallas_call(
        _fused_body,
        grid=grid,
        in_specs=[
            pl.BlockSpec((N, D), lambda i: (0, 0)),
            pl.BlockSpec((R, ROW_TILE, N), lambda i: (0, i, 0)),
            pl.BlockSpec((D, D), lambda i: (0, 0)),
        ],
        out_specs=pl.BlockSpec((ROW_TILE, D), lambda i: (i, 0)),
        out_shape=jax.ShapeDtypeStruct((N, D), jnp.float32),
        scratch_shapes=[pltpu.VMEM((N, D), jnp.float32)],
    )(feature, same_type_adj, W)
